# baseline (speedup 1.0000x reference)
import functools

import jax
import jax.numpy as jnp
from jax import lax
from jax.experimental import pallas as pl
from jax.experimental.pallas import tpu as pltpu, tpu_sc as plsc

B = 16384
D = 64
L = 16
NROWS = 1000000


def _make_sc_kernel():
    info = plsc.get_sparse_core_info()
    nc, ns = info.num_cores, info.num_subcores
    nw = nc * ns
    b_per_w = B // nw
    mesh = plsc.VectorSubcoreMesh(core_axis_name="c", subcore_axis_name="s")

    @functools.partial(
        pl.kernel,
        mesh=mesh,
        compiler_params=pltpu.CompilerParams(use_tc_tiling_on_sc=False),
        out_type=jax.ShapeDtypeStruct((B, D), jnp.float32),
        scratch_types=[
            pltpu.VMEM((b_per_w,), jnp.int32),
            pltpu.VMEM((b_per_w,), jnp.int32),
            pltpu.VMEM((b_per_w,), jnp.int32),
            pltpu.VMEM((b_per_w, D), jnp.float32),
            pltpu.VMEM((b_per_w, D), jnp.float32),
            pltpu.VMEM((b_per_w, D), jnp.float32),
            pltpu.SemaphoreType.DMA,
            pltpu.SemaphoreType.DMA,
            pltpu.SemaphoreType.DMA,
        ],
    )
    def k(pred_idx_hbm, head_idx_hbm, tail_idx_hbm, const_hbm, pred_hbm,
          out_hbm, pidx_v, hidx_v, tidx_v, p_v, h_v, t_v, sem0, sem1, sem2):
        wid = lax.axis_index("s") * nc + lax.axis_index("c")
        base = wid * b_per_w
        pltpu.sync_copy(pred_idx_hbm.at[pl.ds(base, b_per_w)], pidx_v)
        pltpu.sync_copy(head_idx_hbm.at[pl.ds(base, b_per_w)], hidx_v)
        pltpu.sync_copy(tail_idx_hbm.at[pl.ds(base, b_per_w)], tidx_v)
        cp0 = pltpu.async_copy(pred_hbm.at[pidx_v], p_v, sem0)
        cp1 = pltpu.async_copy(const_hbm.at[hidx_v], h_v, sem1)
        cp2 = pltpu.async_copy(const_hbm.at[tidx_v], t_v, sem2)
        cp0.wait()
        cp1.wait()
        cp2.wait()

        def body(i, _):
            for j in range(D // L):
                sl = pl.ds(j * L, L)
                p_v[i, sl] = p_v[i, sl] + h_v[i, sl] - t_v[i, sl]
            return 0

        lax.fori_loop(0, b_per_w, body, 0)
        pltpu.sync_copy(p_v, out_hbm.at[pl.ds(base, b_per_w)])

    return k


_sc_kernel = _make_sc_kernel()


@jax.jit
def kernel(sub_indices, constant_table, predicate_table):
    pred_idx = sub_indices[:, 0]
    head_idx = sub_indices[:, 1]
    tail_idx = sub_indices[:, 2]
    return _sc_kernel(pred_idx, head_idx, tail_idx,
                      constant_table[:NROWS], predicate_table[:NROWS])


# speedup vs baseline: 1.5594x; 1.5594x over previous
import functools

import jax
import jax.numpy as jnp
from jax import lax
from jax.experimental import pallas as pl
from jax.experimental.pallas import tpu as pltpu, tpu_sc as plsc

B = 16384
D = 64
L = 16
CH = 128


def _make_sc_kernel():
    info = plsc.get_sparse_core_info()
    nc, ns = info.num_cores, info.num_subcores
    nw = nc * ns
    b_per_w = B // nw
    n_ch = b_per_w // CH
    mesh = plsc.VectorSubcoreMesh(core_axis_name="c", subcore_axis_name="s")

    @functools.partial(
        pl.kernel,
        mesh=mesh,
        compiler_params=pltpu.CompilerParams(needs_layout_passes=False),
        out_type=jax.ShapeDtypeStruct((B, D), jnp.float32),
        scratch_types=[
            pltpu.VMEM((b_per_w,), jnp.int32),
            pltpu.VMEM((b_per_w,), jnp.int32),
            pltpu.VMEM((b_per_w,), jnp.int32),
            pltpu.VMEM((CH, D), jnp.float32),
            pltpu.VMEM((CH, D), jnp.float32),
            pltpu.VMEM((CH, D), jnp.float32),
            pltpu.SemaphoreType.DMA,
        ],
    )
    def k(pred_idx_hbm, head_idx_hbm, tail_idx_hbm, const_hbm, pred_hbm,
          out_hbm, pidx_v, hidx_v, tidx_v, p_v, h_v, t_v, sem):
        wid = lax.axis_index("s") * nc + lax.axis_index("c")
        base = wid * b_per_w
        pltpu.sync_copy(pred_idx_hbm.at[pl.ds(base, b_per_w)], pidx_v)
        pltpu.sync_copy(head_idx_hbm.at[pl.ds(base, b_per_w)], hidx_v)
        pltpu.sync_copy(tail_idx_hbm.at[pl.ds(base, b_per_w)], tidx_v)

        def chunk_body(ch, _):
            off = pl.multiple_of(ch * CH, 8)

            def issue_body(g, _):
                sl = pl.ds(off + g * L, L)
                for idx_v, tbl, dst in ((pidx_v, pred_hbm, p_v),
                                        (hidx_v, const_hbm, h_v),
                                        (tidx_v, const_hbm, t_v)):
                    vec = idx_v[sl]
                    for j in range(L):
                        r = jnp.squeeze(lax.slice(vec, (j,), (j + 1,)))
                        pltpu.async_copy(tbl.at[r], dst.at[g * L + j], sem)
                return 0

            lax.fori_loop(0, CH // L, issue_body, 0)

            def drain_body(i, _):
                pltpu.make_async_copy(pred_hbm.at[0], p_v.at[0], sem).wait()
                return 0

            lax.fori_loop(0, 3 * CH, drain_body, 0)

            def combine_body(i, _):
                for j in range(D // L):
                    sl = pl.ds(j * L, L)
                    p_v[i, sl] = p_v[i, sl] + h_v[i, sl] - t_v[i, sl]
                return 0

            lax.fori_loop(0, CH, combine_body, 0)
            pltpu.sync_copy(p_v, out_hbm.at[pl.ds(base + off, CH)])
            return 0

        lax.fori_loop(0, n_ch, chunk_body, 0)

    return k


_sc_kernel = _make_sc_kernel()


@jax.jit
def kernel(sub_indices, constant_table, predicate_table):
    pred_idx = sub_indices[:, 0]
    head_idx = sub_indices[:, 1]
    tail_idx = sub_indices[:, 2]
    return _sc_kernel(pred_idx, head_idx, tail_idx, constant_table,
                      predicate_table)
